# baseline (speedup 1.0000x reference)
import functools

import jax
import jax.numpy as jnp
from jax import lax
from jax.experimental import pallas as pl
from jax.experimental.pallas import tpu as pltpu
from jax.experimental.pallas import tpu_sc as plsc

BATCH = 4096
CTX = 50
EMBED_DIM = 64
OUTPUT_DIM = 1000
VOCAB = 1000000

NUM_CORES = 2
NUM_SUBCORES = 16
NUM_WORKERS = NUM_CORES * NUM_SUBCORES
ROWS_PER_WORKER = BATCH // NUM_WORKERS
NLANE = 16
DREG = EMBED_DIM // NLANE
NBUF = 8


_T_BLK = 2048
_ROW_PITCH = 128


def _transpose_body(t_ref, o_ref):
    ident = (
        lax.broadcasted_iota(jnp.int32, (EMBED_DIM, _ROW_PITCH), 0)
        == lax.broadcasted_iota(jnp.int32, (EMBED_DIM, _ROW_PITCH), 1)
    ).astype(jnp.float32)
    o_ref[...] = lax.dot_general(
        t_ref[...], ident,
        dimension_numbers=(((0,), (0,)), ((), ())),
        preferred_element_type=jnp.float32,
        precision=lax.Precision.HIGHEST,
    )


def _relayout_table(table_t):
    grid = (pl.cdiv(VOCAB, _T_BLK),)
    return pl.pallas_call(
        _transpose_body,
        grid=grid,
        in_specs=[pl.BlockSpec((EMBED_DIM, _T_BLK), lambda i: (0, i))],
        out_specs=pl.BlockSpec((_T_BLK, _ROW_PITCH), lambda i: (i, 0)),
        out_shape=jax.ShapeDtypeStruct((VOCAB, _ROW_PITCH), jnp.float32),
    )(table_t)


def _sc_pool(idx_hbm, table_hbm, out_hbm, idx_v, acc_v, *bufs_and_sems):
    bufs = bufs_and_sems[:NBUF]
    sems = bufs_and_sems[NBUF:]
    wid = lax.axis_index("s") * NUM_CORES + lax.axis_index("c")
    base = wid * ROWS_PER_WORKER

    pltpu.sync_copy(idx_hbm.at[pl.ds(base, ROWS_PER_WORKER)], idx_v)

    for b in range(NBUF):
        pltpu.async_copy(table_hbm.at[idx_v.at[b]], bufs[b], sems[b])

    def accumulate(buf, r):
        def body(j, accs):
            return tuple(
                accs[d]
                + buf[2 * j, pl.ds(d * NLANE, NLANE)]
                + buf[2 * j + 1, pl.ds(d * NLANE, NLANE)]
                for d in range(DREG)
            )

        zeros = tuple(jnp.zeros((NLANE,), jnp.float32) for _ in range(DREG))
        accs = lax.fori_loop(0, CTX // 2, body, zeros)
        for d in range(DREG):
            acc_v[r, pl.ds(d * NLANE, NLANE)] = accs[d]

    def group(go, carry):
        for b in range(NBUF):
            r = go * NBUF + b
            pltpu.make_async_copy(
                table_hbm.at[idx_v.at[r]], bufs[b], sems[b]
            ).wait()
            accumulate(bufs[b], r)

            @pl.when(r + NBUF < ROWS_PER_WORKER)
            def _():
                pltpu.async_copy(
                    table_hbm.at[idx_v.at[r + NBUF]], bufs[b], sems[b]
                )

        return carry

    lax.fori_loop(0, ROWS_PER_WORKER // NBUF, group, 0)

    pltpu.sync_copy(acc_v, out_hbm.at[pl.ds(base, ROWS_PER_WORKER)])


def _pool_embeddings(idx, table):
    mesh = plsc.VectorSubcoreMesh(core_axis_name="c", subcore_axis_name="s")
    kern = functools.partial(
        pl.kernel,
        mesh=mesh,
        out_type=jax.ShapeDtypeStruct((BATCH, EMBED_DIM), jnp.float32),
        scratch_types=(
            [
                pltpu.VMEM((ROWS_PER_WORKER, CTX), jnp.int32),
                pltpu.VMEM((ROWS_PER_WORKER, EMBED_DIM), jnp.float32),
            ]
            + [pltpu.VMEM((CTX, _ROW_PITCH), jnp.float32)] * NBUF
            + [pltpu.SemaphoreType.DMA] * NBUF
        ),
        compiler_params=pltpu.CompilerParams(use_tc_tiling_on_sc=False),
    )(_sc_pool)
    return kern(idx, table)


_M_BLK = 512


def _matmul_body(wt_ref, x_ref, bvec_ref, o_ref):
    acc = lax.dot_general(
        wt_ref[...], x_ref[...],
        dimension_numbers=(((0,), (1,)), ((), ())),
        preferred_element_type=jnp.float32,
    )
    o_ref[...] = acc + bvec_ref[...]


def _linear_t(w_t, x, bvec2d):
    grid = (BATCH // _M_BLK,)
    return pl.pallas_call(
        _matmul_body,
        grid=grid,
        in_specs=[
            pl.BlockSpec((EMBED_DIM, OUTPUT_DIM), lambda i: (0, 0)),
            pl.BlockSpec((_M_BLK, EMBED_DIM), lambda i: (i, 0)),
            pl.BlockSpec((OUTPUT_DIM, 1), lambda i: (0, 0)),
        ],
        out_specs=pl.BlockSpec((OUTPUT_DIM, _M_BLK), lambda i: (0, i)),
        out_shape=jax.ShapeDtypeStruct((OUTPUT_DIM, BATCH), jnp.float32),
    )(w_t, x, bvec2d)


def kernel(inputs, embed_table, W, b, bias):
    idx = inputs.astype(jnp.int32)
    table_rm = _relayout_table(jnp.transpose(embed_table))
    pooled = _pool_embeddings(idx, table_rm)
    w_t = jnp.transpose(W)
    bvec2d = (b + bias).astype(jnp.float32).reshape(OUTPUT_DIM, 1)
    logits_t = _linear_t(w_t, pooled, bvec2d)
    return jnp.transpose(logits_t)


# speedup vs baseline: 1.2816x; 1.2816x over previous
import functools

import jax
import jax.numpy as jnp
from jax import lax
from jax.experimental import pallas as pl
from jax.experimental.pallas import tpu as pltpu
from jax.experimental.pallas import tpu_sc as plsc

BATCH = 4096
CTX = 50
EMBED_DIM = 64
OUTPUT_DIM = 1000
VOCAB = 1000000

NUM_CORES = 2
NUM_SUBCORES = 16
NUM_WORKERS = NUM_CORES * NUM_SUBCORES
ROWS_PER_WORKER = BATCH // NUM_WORKERS
NLANE = 16
DREG = EMBED_DIM // NLANE
NBUF = 8


_T_BLK = 2048
_ROW_PITCH = 128


def _transpose_body(t_ref, o_ref):
    o_ref[:, 0:EMBED_DIM] = t_ref[...].T


def _relayout_table(table_t):
    grid = (pl.cdiv(VOCAB, _T_BLK),)
    return pl.pallas_call(
        _transpose_body,
        grid=grid,
        in_specs=[pl.BlockSpec((EMBED_DIM, _T_BLK), lambda i: (0, i))],
        out_specs=pl.BlockSpec((_T_BLK, _ROW_PITCH), lambda i: (i, 0)),
        out_shape=jax.ShapeDtypeStruct((VOCAB, _ROW_PITCH), jnp.float32),
    )(table_t)


def _sc_pool(idx_hbm, table_hbm, out_hbm, idx_v, acc_v, *bufs_and_sems):
    bufs = bufs_and_sems[:NBUF]
    sems = bufs_and_sems[NBUF:]
    wid = lax.axis_index("s") * NUM_CORES + lax.axis_index("c")
    base = wid * ROWS_PER_WORKER

    pltpu.sync_copy(idx_hbm.at[pl.ds(base, ROWS_PER_WORKER)], idx_v)

    for b in range(NBUF):
        pltpu.async_copy(table_hbm.at[idx_v.at[b]], bufs[b], sems[b])

    def accumulate(buf, r):
        def body(j, accs):
            return tuple(
                accs[d]
                + buf[2 * j, pl.ds(d * NLANE, NLANE)]
                + buf[2 * j + 1, pl.ds(d * NLANE, NLANE)]
                for d in range(DREG)
            )

        zeros = tuple(jnp.zeros((NLANE,), jnp.float32) for _ in range(DREG))
        accs = lax.fori_loop(0, CTX // 2, body, zeros)
        for d in range(DREG):
            acc_v[r, pl.ds(d * NLANE, NLANE)] = accs[d]

    def group(go, carry):
        for b in range(NBUF):
            r = go * NBUF + b
            pltpu.make_async_copy(
                table_hbm.at[idx_v.at[r]], bufs[b], sems[b]
            ).wait()
            accumulate(bufs[b], r)

            @pl.when(r + NBUF < ROWS_PER_WORKER)
            def _():
                pltpu.async_copy(
                    table_hbm.at[idx_v.at[r + NBUF]], bufs[b], sems[b]
                )

        return carry

    lax.fori_loop(0, ROWS_PER_WORKER // NBUF, group, 0)

    pltpu.sync_copy(acc_v, out_hbm.at[pl.ds(base, ROWS_PER_WORKER)])


def _pool_embeddings(idx, table):
    mesh = plsc.VectorSubcoreMesh(core_axis_name="c", subcore_axis_name="s")
    kern = functools.partial(
        pl.kernel,
        mesh=mesh,
        out_type=jax.ShapeDtypeStruct((BATCH, EMBED_DIM), jnp.float32),
        scratch_types=(
            [
                pltpu.VMEM((ROWS_PER_WORKER, CTX), jnp.int32),
                pltpu.VMEM((ROWS_PER_WORKER, EMBED_DIM), jnp.float32),
            ]
            + [pltpu.VMEM((CTX, _ROW_PITCH), jnp.float32)] * NBUF
            + [pltpu.SemaphoreType.DMA] * NBUF
        ),
        compiler_params=pltpu.CompilerParams(use_tc_tiling_on_sc=False),
    )(_sc_pool)
    return kern(idx, table)


_M_BLK = 512


def _matmul_body(wt_ref, x_ref, bvec_ref, o_ref):
    acc = lax.dot_general(
        wt_ref[...], x_ref[...],
        dimension_numbers=(((0,), (1,)), ((), ())),
        preferred_element_type=jnp.float32,
    )
    o_ref[...] = acc + bvec_ref[...]


def _linear_t(w_t, x, bvec2d):
    grid = (BATCH // _M_BLK,)
    return pl.pallas_call(
        _matmul_body,
        grid=grid,
        in_specs=[
            pl.BlockSpec((EMBED_DIM, OUTPUT_DIM), lambda i: (0, 0)),
            pl.BlockSpec((_M_BLK, EMBED_DIM), lambda i: (i, 0)),
            pl.BlockSpec((OUTPUT_DIM, 1), lambda i: (0, 0)),
        ],
        out_specs=pl.BlockSpec((OUTPUT_DIM, _M_BLK), lambda i: (0, i)),
        out_shape=jax.ShapeDtypeStruct((OUTPUT_DIM, BATCH), jnp.float32),
    )(w_t, x, bvec2d)


def kernel(inputs, embed_table, W, b, bias):
    idx = inputs.astype(jnp.int32)
    table_rm = _relayout_table(jnp.transpose(embed_table))
    pooled = _pool_embeddings(idx, table_rm)
    w_t = jnp.transpose(W)
    bvec2d = (b + bias).astype(jnp.float32).reshape(OUTPUT_DIM, 1)
    logits_t = _linear_t(w_t, pooled, bvec2d)
    return jnp.transpose(logits_t)


# speedup vs baseline: 1.3247x; 1.0336x over previous
import functools

import jax
import jax.numpy as jnp
from jax import lax
from jax.experimental import pallas as pl
from jax.experimental.pallas import tpu as pltpu
from jax.experimental.pallas import tpu_sc as plsc

BATCH = 4096
CTX = 50
EMBED_DIM = 64
OUTPUT_DIM = 1000
VOCAB = 1000000

HALF = 1 << 19

NUM_CORES = 2
NUM_SUBCORES = 16
NUM_WORKERS = NUM_CORES * NUM_SUBCORES
ROWS_PER_WORKER = BATCH // NUM_WORKERS
NLANE = 16
DREG = EMBED_DIM // NLANE
NBUF = 8


_T_BLK = 1024


def _transpose_body(a_ref, b_ref, o_ref):
    o_ref[:, 0:EMBED_DIM] = a_ref[...].T
    o_ref[:, EMBED_DIM : 2 * EMBED_DIM] = b_ref[...].T


def _relayout_table(table_t):
    nb = HALF // _T_BLK
    return pl.pallas_call(
        _transpose_body,
        grid=(nb,),
        in_specs=[
            pl.BlockSpec((EMBED_DIM, _T_BLK), lambda i: (0, i)),
            pl.BlockSpec(
                (EMBED_DIM, _T_BLK),
                lambda i: (0, jnp.minimum(i + HALF // _T_BLK,
                                          pl.cdiv(VOCAB, _T_BLK) - 1)),
            ),
        ],
        out_specs=pl.BlockSpec((_T_BLK, 2 * EMBED_DIM), lambda i: (i, 0)),
        out_shape=jax.ShapeDtypeStruct((HALF, 2 * EMBED_DIM), jnp.float32),
    )(table_t, table_t)


def _sc_pool(idx_hbm, table_hbm, out_hbm, idx_v, acc_v, *bufs_and_sems):
    bufs = bufs_and_sems[:NBUF]
    sems = bufs_and_sems[NBUF:]
    wid = lax.axis_index("s") * NUM_CORES + lax.axis_index("c")
    base = wid * ROWS_PER_WORKER

    pltpu.sync_copy(idx_hbm.at[pl.ds(base, ROWS_PER_WORKER)], idx_v)

    for b in range(NBUF):
        pltpu.async_copy(table_hbm.at[idx_v.at[b]], bufs[b], sems[b])

    def accumulate(buf, r):
        def body(j, accs):
            return tuple(
                accs[d]
                + buf[2 * j, pl.ds(d * NLANE, NLANE)]
                + buf[2 * j + 1, pl.ds(d * NLANE, NLANE)]
                for d in range(DREG)
            )

        zeros = tuple(jnp.zeros((NLANE,), jnp.float32) for _ in range(DREG))
        accs = lax.fori_loop(0, CTX // 2, body, zeros)
        for d in range(DREG):
            acc_v[r, pl.ds(d * NLANE, NLANE)] = accs[d]

    def group(go, carry):
        for b in range(NBUF):
            r = go * NBUF + b
            pltpu.make_async_copy(
                table_hbm.at[idx_v.at[r]], bufs[b], sems[b]
            ).wait()
            accumulate(bufs[b], r)

            @pl.when(r + NBUF < ROWS_PER_WORKER)
            def _():
                pltpu.async_copy(
                    table_hbm.at[idx_v.at[r + NBUF]], bufs[b], sems[b]
                )

        return carry

    lax.fori_loop(0, ROWS_PER_WORKER // NBUF, group, 0)

    pltpu.sync_copy(acc_v, out_hbm.at[pl.ds(base, ROWS_PER_WORKER)])


def _pool_embeddings(gidx, table):
    mesh = plsc.VectorSubcoreMesh(core_axis_name="c", subcore_axis_name="s")
    kern = functools.partial(
        pl.kernel,
        mesh=mesh,
        out_type=jax.ShapeDtypeStruct((BATCH, EMBED_DIM), jnp.float32),
        scratch_types=(
            [
                pltpu.VMEM((ROWS_PER_WORKER, CTX), jnp.int32),
                pltpu.VMEM((ROWS_PER_WORKER, EMBED_DIM), jnp.float32),
            ]
            + [pltpu.VMEM((CTX, EMBED_DIM), jnp.float32)] * NBUF
            + [pltpu.SemaphoreType.DMA] * NBUF
        ),
        compiler_params=pltpu.CompilerParams(use_tc_tiling_on_sc=False),
    )(_sc_pool)
    return kern(gidx, table)


_M_BLK = 512


def _matmul_body(wt_ref, x_ref, bvec_ref, o_ref):
    acc = lax.dot_general(
        wt_ref[...], x_ref[...],
        dimension_numbers=(((0,), (1,)), ((), ())),
        preferred_element_type=jnp.float32,
    )
    o_ref[...] = acc + bvec_ref[...]


def _linear_t(w_t, x, bvec2d):
    grid = (BATCH // _M_BLK,)
    return pl.pallas_call(
        _matmul_body,
        grid=grid,
        in_specs=[
            pl.BlockSpec((EMBED_DIM, OUTPUT_DIM), lambda i: (0, 0)),
            pl.BlockSpec((_M_BLK, EMBED_DIM), lambda i: (i, 0)),
            pl.BlockSpec((OUTPUT_DIM, 1), lambda i: (0, 0)),
        ],
        out_specs=pl.BlockSpec((OUTPUT_DIM, _M_BLK), lambda i: (0, i)),
        out_shape=jax.ShapeDtypeStruct((OUTPUT_DIM, BATCH), jnp.float32),
    )(w_t, x, bvec2d)


def kernel(inputs, embed_table, W, b, bias):
    idx = inputs.astype(jnp.int32)
    gidx = 2 * jnp.bitwise_and(idx, HALF - 1) + (idx >= HALF).astype(jnp.int32)
    table_pk = _relayout_table(jnp.transpose(embed_table))
    table_rows = table_pk.reshape(2 * HALF, EMBED_DIM)
    pooled = _pool_embeddings(gidx, table_rows)
    w_t = jnp.transpose(W)
    bvec2d = (b + bias).astype(jnp.float32).reshape(OUTPUT_DIM, 1)
    logits_t = _linear_t(w_t, pooled, bvec2d)
    return jnp.transpose(logits_t)


# speedup vs baseline: 1.7639x; 1.3316x over previous
import functools

import jax
import jax.numpy as jnp
from jax import lax
from jax.experimental import pallas as pl
from jax.experimental.pallas import tpu as pltpu
from jax.experimental.pallas import tpu_sc as plsc

BATCH = 4096
CTX = 50
EMBED_DIM = 64
OUTPUT_DIM = 1000
VOCAB = 1000000

HALF = 1 << 19

NUM_CORES = 2
NUM_SUBCORES = 16
NUM_WORKERS = NUM_CORES * NUM_SUBCORES
ROWS_PER_WORKER = BATCH // NUM_WORKERS
NLANE = 16
DREG = EMBED_DIM // NLANE
NBUF = 8


_T_BLK = 2048


def _transpose_body(a_ref, b_ref, o_ref):
    o_ref[:, 0:EMBED_DIM] = a_ref[...].T
    o_ref[:, EMBED_DIM : 2 * EMBED_DIM] = b_ref[...].T


def _relayout_table(table_t):
    nb = HALF // _T_BLK
    return pl.pallas_call(
        _transpose_body,
        grid=(nb,),
        in_specs=[
            pl.BlockSpec((EMBED_DIM, _T_BLK), lambda i: (0, i)),
            pl.BlockSpec(
                (EMBED_DIM, _T_BLK),
                lambda i: (0, jnp.minimum(i + HALF // _T_BLK,
                                          pl.cdiv(VOCAB, _T_BLK) - 1)),
            ),
        ],
        out_specs=pl.BlockSpec((_T_BLK, 2 * EMBED_DIM), lambda i: (i, 0)),
        out_shape=jax.ShapeDtypeStruct((HALF, 2 * EMBED_DIM), jnp.float32),
    )(table_t, table_t)


def _sc_pool(idx_hbm, table_hbm, out_hbm, idx_v, acc_v, *bufs_and_sems):
    bufs = bufs_and_sems[:NBUF]
    sems = bufs_and_sems[NBUF:]
    wid = lax.axis_index("s") * NUM_CORES + lax.axis_index("c")
    base = wid * ROWS_PER_WORKER

    pltpu.sync_copy(idx_hbm.at[pl.ds(base, ROWS_PER_WORKER)], idx_v)

    for b in range(NBUF):
        pltpu.async_copy(table_hbm.at[idx_v.at[b]], bufs[b], sems[b])

    def accumulate(buf, r):
        def body(j, accs):
            return tuple(
                accs[d]
                + buf[2 * j, pl.ds(d * NLANE, NLANE)]
                + buf[2 * j + 1, pl.ds(d * NLANE, NLANE)]
                for d in range(DREG)
            )

        zeros = tuple(jnp.zeros((NLANE,), jnp.float32) for _ in range(DREG))
        accs = lax.fori_loop(0, CTX // 2, body, zeros)
        for d in range(DREG):
            acc_v[r, pl.ds(d * NLANE, NLANE)] = accs[d]

    def group(go, carry):
        for b in range(NBUF):
            r = go * NBUF + b
            pltpu.make_async_copy(
                table_hbm.at[idx_v.at[r]], bufs[b], sems[b]
            ).wait()
            accumulate(bufs[b], r)

            @pl.when(r + NBUF < ROWS_PER_WORKER)
            def _():
                pltpu.async_copy(
                    table_hbm.at[idx_v.at[r + NBUF]], bufs[b], sems[b]
                )

        return carry

    lax.fori_loop(0, ROWS_PER_WORKER // NBUF, group, 0)

    pltpu.sync_copy(acc_v, out_hbm.at[pl.ds(base, ROWS_PER_WORKER)])


def _pool_embeddings(gidx, table):
    mesh = plsc.VectorSubcoreMesh(core_axis_name="c", subcore_axis_name="s")
    kern = functools.partial(
        pl.kernel,
        mesh=mesh,
        out_type=jax.ShapeDtypeStruct((BATCH, EMBED_DIM), jnp.float32),
        scratch_types=(
            [
                pltpu.VMEM((ROWS_PER_WORKER, CTX), jnp.int32),
                pltpu.VMEM((ROWS_PER_WORKER, EMBED_DIM), jnp.float32),
            ]
            + [pltpu.VMEM((CTX, EMBED_DIM), jnp.float32)] * NBUF
            + [pltpu.SemaphoreType.DMA] * NBUF
        ),
        compiler_params=pltpu.CompilerParams(use_tc_tiling_on_sc=False),
    )(_sc_pool)
    return kern(gidx, table)


_M_BLK = 512


def _matmul_body(wt_ref, x_ref, bvec_ref, o_ref):
    acc = lax.dot_general(
        wt_ref[...], x_ref[...],
        dimension_numbers=(((0,), (1,)), ((), ())),
        preferred_element_type=jnp.float32,
    )
    o_ref[...] = acc + bvec_ref[...]


def _linear_t(w_t, x, bvec2d):
    grid = (BATCH // _M_BLK,)
    return pl.pallas_call(
        _matmul_body,
        grid=grid,
        in_specs=[
            pl.BlockSpec((EMBED_DIM, OUTPUT_DIM), lambda i: (0, 0)),
            pl.BlockSpec((_M_BLK, EMBED_DIM), lambda i: (i, 0)),
            pl.BlockSpec((OUTPUT_DIM, 1), lambda i: (0, 0)),
        ],
        out_specs=pl.BlockSpec((OUTPUT_DIM, _M_BLK), lambda i: (0, i)),
        out_shape=jax.ShapeDtypeStruct((OUTPUT_DIM, BATCH), jnp.float32),
    )(w_t, x, bvec2d)


def kernel(inputs, embed_table, W, b, bias):
    idx = inputs.astype(jnp.int32)
    gidx = 2 * jnp.bitwise_and(idx, HALF - 1) + (idx >= HALF).astype(jnp.int32)
    table_pk = _relayout_table(jnp.transpose(embed_table))
    table_rows = table_pk.reshape(2 * HALF, EMBED_DIM)
    pooled = _pool_embeddings(gidx, table_rows)
    w_t = jnp.transpose(W)
    bvec2d = (b + bias).astype(jnp.float32).reshape(OUTPUT_DIM, 1)
    logits_t = _linear_t(w_t, pooled, bvec2d)
    return jnp.transpose(logits_t)


# speedup vs baseline: 2.1407x; 1.2136x over previous
import functools

import jax
import jax.numpy as jnp
from jax import lax
from jax.experimental import pallas as pl
from jax.experimental.pallas import tpu as pltpu
from jax.experimental.pallas import tpu_sc as plsc

BATCH = 4096
CTX = 50
EMBED_DIM = 64
OUTPUT_DIM = 1000
VOCAB = 1000000

HALF = 1 << 19

NUM_CORES = 2
NUM_SUBCORES = 16
NUM_WORKERS = NUM_CORES * NUM_SUBCORES
ROWS_PER_WORKER = BATCH // NUM_WORKERS
NLANE = 16
DREG = EMBED_DIM // NLANE
NBUF = 8


_T_BLK = 4096


def _transpose_body(a_ref, b_ref, o_ref):
    o_ref[:, 0:EMBED_DIM] = a_ref[...].T
    o_ref[:, EMBED_DIM : 2 * EMBED_DIM] = b_ref[...].T


def _relayout_table(table_t):
    nb = HALF // _T_BLK
    return pl.pallas_call(
        _transpose_body,
        grid=(nb,),
        in_specs=[
            pl.BlockSpec((EMBED_DIM, _T_BLK), lambda i: (0, i)),
            pl.BlockSpec(
                (EMBED_DIM, _T_BLK),
                lambda i: (0, jnp.minimum(i + HALF // _T_BLK,
                                          pl.cdiv(VOCAB, _T_BLK) - 1)),
            ),
        ],
        out_specs=pl.BlockSpec((_T_BLK, 2 * EMBED_DIM), lambda i: (i, 0)),
        out_shape=jax.ShapeDtypeStruct((HALF, 2 * EMBED_DIM), jnp.float32),
    )(table_t, table_t)


def _sc_pool(idx_hbm, table_hbm, out_hbm, idx_v, acc_v, *bufs_and_sems):
    bufs = bufs_and_sems[:NBUF]
    sems = bufs_and_sems[NBUF:]
    wid = lax.axis_index("s") * NUM_CORES + lax.axis_index("c")
    base = wid * ROWS_PER_WORKER

    pltpu.sync_copy(idx_hbm.at[pl.ds(base, ROWS_PER_WORKER)], idx_v)

    for b in range(NBUF):
        pltpu.async_copy(table_hbm.at[idx_v.at[b]], bufs[b], sems[b])

    def accumulate(buf, r):
        def body(j, accs):
            return tuple(
                accs[d]
                + buf[2 * j, pl.ds(d * NLANE, NLANE)]
                + buf[2 * j + 1, pl.ds(d * NLANE, NLANE)]
                for d in range(DREG)
            )

        zeros = tuple(jnp.zeros((NLANE,), jnp.float32) for _ in range(DREG))
        accs = lax.fori_loop(0, CTX // 2, body, zeros)
        for d in range(DREG):
            acc_v[r, pl.ds(d * NLANE, NLANE)] = accs[d]

    def group(go, carry):
        for b in range(NBUF):
            r = go * NBUF + b
            pltpu.make_async_copy(
                table_hbm.at[idx_v.at[r]], bufs[b], sems[b]
            ).wait()
            accumulate(bufs[b], r)

            @pl.when(r + NBUF < ROWS_PER_WORKER)
            def _():
                pltpu.async_copy(
                    table_hbm.at[idx_v.at[r + NBUF]], bufs[b], sems[b]
                )

        return carry

    lax.fori_loop(0, ROWS_PER_WORKER // NBUF, group, 0)

    pltpu.sync_copy(acc_v, out_hbm.at[pl.ds(base, ROWS_PER_WORKER)])


def _pool_embeddings(gidx, table):
    mesh = plsc.VectorSubcoreMesh(core_axis_name="c", subcore_axis_name="s")
    kern = functools.partial(
        pl.kernel,
        mesh=mesh,
        out_type=jax.ShapeDtypeStruct((BATCH, EMBED_DIM), jnp.float32),
        scratch_types=(
            [
                pltpu.VMEM((ROWS_PER_WORKER, CTX), jnp.int32),
                pltpu.VMEM((ROWS_PER_WORKER, EMBED_DIM), jnp.float32),
            ]
            + [pltpu.VMEM((CTX, EMBED_DIM), jnp.float32)] * NBUF
            + [pltpu.SemaphoreType.DMA] * NBUF
        ),
        compiler_params=pltpu.CompilerParams(use_tc_tiling_on_sc=False),
    )(_sc_pool)
    return kern(gidx, table)


_M_BLK = 512


def _matmul_body(wt_ref, x_ref, bvec_ref, o_ref):
    acc = lax.dot_general(
        wt_ref[...], x_ref[...],
        dimension_numbers=(((0,), (1,)), ((), ())),
        preferred_element_type=jnp.float32,
    )
    o_ref[...] = acc + bvec_ref[...]


def _linear_t(w_t, x, bvec2d):
    grid = (BATCH // _M_BLK,)
    return pl.pallas_call(
        _matmul_body,
        grid=grid,
        in_specs=[
            pl.BlockSpec((EMBED_DIM, OUTPUT_DIM), lambda i: (0, 0)),
            pl.BlockSpec((_M_BLK, EMBED_DIM), lambda i: (i, 0)),
            pl.BlockSpec((OUTPUT_DIM, 1), lambda i: (0, 0)),
        ],
        out_specs=pl.BlockSpec((OUTPUT_DIM, _M_BLK), lambda i: (0, i)),
        out_shape=jax.ShapeDtypeStruct((OUTPUT_DIM, BATCH), jnp.float32),
    )(w_t, x, bvec2d)


def kernel(inputs, embed_table, W, b, bias):
    idx = inputs.astype(jnp.int32)
    gidx = 2 * jnp.bitwise_and(idx, HALF - 1) + (idx >= HALF).astype(jnp.int32)
    table_pk = _relayout_table(jnp.transpose(embed_table))
    table_rows = table_pk.reshape(2 * HALF, EMBED_DIM)
    pooled = _pool_embeddings(gidx, table_rows)
    w_t = jnp.transpose(W)
    bvec2d = (b + bias).astype(jnp.float32).reshape(OUTPUT_DIM, 1)
    logits_t = _linear_t(w_t, pooled, bvec2d)
    return jnp.transpose(logits_t)


# speedup vs baseline: 2.4004x; 1.1213x over previous
import functools

import jax
import jax.numpy as jnp
from jax import lax
from jax.experimental import pallas as pl
from jax.experimental.pallas import tpu as pltpu
from jax.experimental.pallas import tpu_sc as plsc

BATCH = 4096
CTX = 50
EMBED_DIM = 64
OUTPUT_DIM = 1000
VOCAB = 1000000

HALF = 1 << 19

NUM_CORES = 2
NUM_SUBCORES = 16
NUM_WORKERS = NUM_CORES * NUM_SUBCORES
ROWS_PER_WORKER = BATCH // NUM_WORKERS
NLANE = 16
DREG = EMBED_DIM // NLANE
NBUF = 8


_T_BLK = 8192


def _transpose_body(a_ref, b_ref, o_ref):
    o_ref[:, 0:EMBED_DIM] = a_ref[...].T
    o_ref[:, EMBED_DIM : 2 * EMBED_DIM] = b_ref[...].T


def _relayout_table(table_t):
    nb = HALF // _T_BLK
    return pl.pallas_call(
        _transpose_body,
        grid=(nb,),
        in_specs=[
            pl.BlockSpec((EMBED_DIM, _T_BLK), lambda i: (0, i)),
            pl.BlockSpec(
                (EMBED_DIM, _T_BLK),
                lambda i: (0, jnp.minimum(i + HALF // _T_BLK,
                                          pl.cdiv(VOCAB, _T_BLK) - 1)),
            ),
        ],
        out_specs=pl.BlockSpec((_T_BLK, 2 * EMBED_DIM), lambda i: (i, 0)),
        out_shape=jax.ShapeDtypeStruct((HALF, 2 * EMBED_DIM), jnp.float32),
    )(table_t, table_t)


def _sc_pool(idx_hbm, table_hbm, out_hbm, idx_v, acc_v, *bufs_and_sems):
    bufs = bufs_and_sems[:NBUF]
    sems = bufs_and_sems[NBUF:]
    wid = lax.axis_index("s") * NUM_CORES + lax.axis_index("c")
    base = wid * ROWS_PER_WORKER

    pltpu.sync_copy(idx_hbm.at[pl.ds(base, ROWS_PER_WORKER)], idx_v)

    for b in range(NBUF):
        pltpu.async_copy(table_hbm.at[idx_v.at[b]], bufs[b], sems[b])

    def accumulate(buf, r):
        def body(j, accs):
            return tuple(
                accs[d]
                + buf[2 * j, pl.ds(d * NLANE, NLANE)]
                + buf[2 * j + 1, pl.ds(d * NLANE, NLANE)]
                for d in range(DREG)
            )

        zeros = tuple(jnp.zeros((NLANE,), jnp.float32) for _ in range(DREG))
        accs = lax.fori_loop(0, CTX // 2, body, zeros)
        for d in range(DREG):
            acc_v[r, pl.ds(d * NLANE, NLANE)] = accs[d]

    def group(go, carry):
        for b in range(NBUF):
            r = go * NBUF + b
            pltpu.make_async_copy(
                table_hbm.at[idx_v.at[r]], bufs[b], sems[b]
            ).wait()
            accumulate(bufs[b], r)

            @pl.when(r + NBUF < ROWS_PER_WORKER)
            def _():
                pltpu.async_copy(
                    table_hbm.at[idx_v.at[r + NBUF]], bufs[b], sems[b]
                )

        return carry

    lax.fori_loop(0, ROWS_PER_WORKER // NBUF, group, 0)

    pltpu.sync_copy(acc_v, out_hbm.at[pl.ds(base, ROWS_PER_WORKER)])


def _pool_embeddings(gidx, table):
    mesh = plsc.VectorSubcoreMesh(core_axis_name="c", subcore_axis_name="s")
    kern = functools.partial(
        pl.kernel,
        mesh=mesh,
        out_type=jax.ShapeDtypeStruct((BATCH, EMBED_DIM), jnp.float32),
        scratch_types=(
            [
                pltpu.VMEM((ROWS_PER_WORKER, CTX), jnp.int32),
                pltpu.VMEM((ROWS_PER_WORKER, EMBED_DIM), jnp.float32),
            ]
            + [pltpu.VMEM((CTX, EMBED_DIM), jnp.float32)] * NBUF
            + [pltpu.SemaphoreType.DMA] * NBUF
        ),
        compiler_params=pltpu.CompilerParams(use_tc_tiling_on_sc=False),
    )(_sc_pool)
    return kern(gidx, table)


_M_BLK = 512


def _matmul_body(wt_ref, x_ref, bvec_ref, o_ref):
    acc = lax.dot_general(
        wt_ref[...], x_ref[...],
        dimension_numbers=(((0,), (1,)), ((), ())),
        preferred_element_type=jnp.float32,
    )
    o_ref[...] = acc + bvec_ref[...]


def _linear_t(w_t, x, bvec2d):
    grid = (BATCH // _M_BLK,)
    return pl.pallas_call(
        _matmul_body,
        grid=grid,
        in_specs=[
            pl.BlockSpec((EMBED_DIM, OUTPUT_DIM), lambda i: (0, 0)),
            pl.BlockSpec((_M_BLK, EMBED_DIM), lambda i: (i, 0)),
            pl.BlockSpec((OUTPUT_DIM, 1), lambda i: (0, 0)),
        ],
        out_specs=pl.BlockSpec((OUTPUT_DIM, _M_BLK), lambda i: (0, i)),
        out_shape=jax.ShapeDtypeStruct((OUTPUT_DIM, BATCH), jnp.float32),
    )(w_t, x, bvec2d)


def kernel(inputs, embed_table, W, b, bias):
    idx = inputs.astype(jnp.int32)
    gidx = 2 * jnp.bitwise_and(idx, HALF - 1) + (idx >= HALF).astype(jnp.int32)
    table_pk = _relayout_table(jnp.transpose(embed_table))
    table_rows = table_pk.reshape(2 * HALF, EMBED_DIM)
    pooled = _pool_embeddings(gidx, table_rows)
    w_t = jnp.transpose(W)
    bvec2d = (b + bias).astype(jnp.float32).reshape(OUTPUT_DIM, 1)
    logits_t = _linear_t(w_t, pooled, bvec2d)
    return jnp.transpose(logits_t)
